# Initial kernel scaffold; baseline (speedup 1.0000x reference)
import jax
import jax.numpy as jnp
from jax.experimental import pallas as pl


def kernel(u_weight, v_weight, pos_u, pos_v, neg_v):
    raise NotImplementedError("write your pallas kernel here")


# speedup vs baseline: 4.3511x; 4.3511x over previous
import functools

import jax
import jax.numpy as jnp
from jax import lax
from jax.experimental import pallas as pl
from jax.experimental.pallas import tpu as pltpu
from jax.experimental.pallas import tpu_sc as plsc

VOCAB = 100000
DIM = 128
BATCH = 16384
NEG = 20

NC = 2
NS = 16
L = 16
NW = NC * NS
BPW = BATCH // NW
CHUNK = L
NCHUNK = BPW // CHUNK
NEGC = CHUNK * NEG
DC = DIM // L


def _sc_scores(u_weight, v_weight, pos_u, pos_v, neg_idx_flat):
  mesh = plsc.VectorSubcoreMesh(
      core_axis_name="c", subcore_axis_name="s", num_cores=NC,
      num_subcores=NS)

  @functools.partial(
      pl.kernel,
      out_type=[
          jax.ShapeDtypeStruct((BATCH,), jnp.float32),
          jax.ShapeDtypeStruct((BATCH * NEG,), jnp.float32),
      ],
      mesh=mesh,
      scratch_types=[
          pltpu.VMEM((BPW,), jnp.int32),
          pltpu.VMEM((BPW,), jnp.int32),
          pltpu.VMEM((BPW * NEG,), jnp.int32),
          pltpu.VMEM((CHUNK, DIM), jnp.float32),
          pltpu.VMEM((CHUNK, DIM), jnp.float32),
          pltpu.VMEM((NEGC, DIM), jnp.float32),
          pltpu.VMEM((CHUNK * L,), jnp.float32),
          pltpu.VMEM((NEGC * L,), jnp.float32),
          pltpu.VMEM((BPW,), jnp.float32),
          pltpu.VMEM((BPW * NEG,), jnp.float32),
          pltpu.SemaphoreType.DMA,
      ],
      compiler_params=pltpu.CompilerParams(needs_layout_passes=False),
  )
  def kern(u_hbm, v_hbm, pu_hbm, pv_hbm, nv_hbm, pos_out, neg_out,
           idx_u, idx_v, idx_neg, u_rows, v_rows, neg_rows,
           pstage, nstage, pos_all, neg_all, sem):
    wid = lax.axis_index("s") * NC + lax.axis_index("c")
    base = wid * BPW

    pltpu.sync_copy(pu_hbm.at[pl.ds(base, BPW)], idx_u)
    pltpu.sync_copy(pv_hbm.at[pl.ds(base, BPW)], idx_v)
    pltpu.sync_copy(nv_hbm.at[pl.ds(base * NEG, BPW * NEG)], idx_neg)

    lanes = lax.iota(jnp.int32, L)

    def chunk_body(c, carry):
      cbase = c * CHUNK
      d1 = pltpu.async_copy(u_hbm.at[idx_u.at[pl.ds(cbase, CHUNK)]],
                            u_rows, sem)
      d2 = pltpu.async_copy(v_hbm.at[idx_v.at[pl.ds(cbase, CHUNK)]],
                            v_rows, sem)
      nbase = cbase * NEG
      d3 = pltpu.async_copy(v_hbm.at[idx_neg.at[pl.ds(nbase, 128)]],
                            neg_rows.at[pl.ds(0, 128)], sem)
      d4 = pltpu.async_copy(v_hbm.at[idx_neg.at[pl.ds(nbase + 128, 128)]],
                            neg_rows.at[pl.ds(128, 128)], sem)
      d5 = pltpu.async_copy(v_hbm.at[idx_neg.at[pl.ds(nbase + 256, 64)]],
                            neg_rows.at[pl.ds(256, 64)], sem)
      d1.wait(); d2.wait(); d3.wait(); d4.wait(); d5.wait()

      def item_body(i, carry2):
        u = [u_rows[i, pl.ds(cc * L, L)] for cc in range(DC)]
        accp = u[0] * v_rows[i, pl.ds(0, L)]
        for cc in range(1, DC):
          accp = accp + u[cc] * v_rows[i, pl.ds(cc * L, L)]
        pstage[pl.ds(i * L, L)] = accp
        for kk in range(NEG):
          r = i * NEG + kk
          accn = u[0] * neg_rows[r, pl.ds(0, L)]
          for cc in range(1, DC):
            accn = accn + u[cc] * neg_rows[r, pl.ds(cc * L, L)]
          nstage[pl.ds(r * L, L)] = accn
        return carry2

      lax.fori_loop(0, CHUNK, item_body, 0)

      sp = plsc.load_gather(pstage, [lanes * L])
      for l in range(1, L):
        sp = sp + plsc.load_gather(pstage, [lanes * L + l])
      pos_all[pl.ds(cbase, CHUNK)] = sp

      for kk in range(NEG):
        ridx = (lanes * NEG + kk) * L
        sn = plsc.load_gather(nstage, [ridx])
        for l in range(1, L):
          sn = sn + plsc.load_gather(nstage, [ridx + l])
        plsc.store_scatter(neg_all, [nbase + lanes * NEG + kk], sn)
      return carry

    lax.fori_loop(0, NCHUNK, chunk_body, 0)

    pltpu.sync_copy(pos_all, pos_out.at[pl.ds(base, BPW)])
    pltpu.sync_copy(neg_all, neg_out.at[pl.ds(base * NEG, BPW * NEG)])

  return kern(u_weight, v_weight, pos_u, pos_v, neg_idx_flat)


def _loss_epilogue(pos_score, neg_score):
  pos2 = pos_score.reshape(BATCH // DIM, DIM)
  neg2 = neg_score.reshape(BATCH * NEG // DIM, DIM)

  def body(p_ref, n_ref, o_ref):
    p = jnp.clip(p_ref[...], -10.0, 10.0)
    n = jnp.clip(n_ref[...], -10.0, 10.0)
    tot = jnp.sum(jnp.log(1.0 + jnp.exp(-p))) + jnp.sum(
        jnp.log(1.0 + jnp.exp(n)))
    o_ref[0, 0] = tot / BATCH

  out = pl.pallas_call(
      body,
      out_shape=jax.ShapeDtypeStruct((1, 1), jnp.float32),
      out_specs=pl.BlockSpec(memory_space=pltpu.SMEM),
  )(pos2, neg2)
  return out[0, 0]


def kernel(u_weight, v_weight, pos_u, pos_v, neg_v):
  pos_u = pos_u.astype(jnp.int32)
  pos_v = pos_v.astype(jnp.int32)
  neg_flat = neg_v.astype(jnp.int32).reshape(-1)
  pos_score, neg_score = _sc_scores(u_weight, v_weight, pos_u, pos_v,
                                    neg_flat)
  return _loss_epilogue(pos_score, neg_score)
